# Initial kernel scaffold; baseline (speedup 1.0000x reference)
import functools

import jax
import jax.numpy as jnp
from jax import lax
from jax.experimental import pallas as pl
from jax.experimental.pallas import tpu as pltpu
from jax.experimental.pallas import tpu_sc as plsc

N = 10000
E = 160000
D = 256

NC, NS = 2, 16
NP = 10240
H = NP // NC
EP = 163840
EW = EP // (NC * NS)
ES = EP // NS
G = 128
SCH = 2048
RPS = H // NS
CAP = ES + 2 * G
PAD_ROW = N + 100
PAD_COL = N

_MESH = plsc.VectorSubcoreMesh(core_axis_name="c", subcore_axis_name="s")


def _sc_degree(rows):
    @functools.partial(
        pl.kernel,
        out_type=jax.ShapeDtypeStruct((NC, NP, 16), jnp.float32),
        mesh=_MESH,
        scratch_types=[
            pltpu.VMEM((EW,), jnp.int32),
            pltpu.VMEM((EW // G, G), jnp.int32),
            pltpu.VMEM((G, 16), jnp.float32),
            pltpu.VMEM((NP // NS, 16), jnp.float32),
            pltpu.VMEM_SHARED((NP, 16), jnp.float32),
        ],
    )
    def deg_kernel(rows_hbm, out_hbm, idx1d, idx2d, ones, zbuf, hist):
        c = lax.axis_index("c")
        s = lax.axis_index("s")
        w = c * NS + s
        ones16 = jnp.ones((16,), jnp.float32)
        zeros16 = jnp.zeros((16,), jnp.float32)

        @pl.loop(0, G)
        def _(i):
            ones[i, :] = ones16

        @pl.loop(0, NP // NS)
        def _(i):
            zbuf[i, :] = zeros16

        pltpu.sync_copy(rows_hbm.at[pl.ds(w * EW, EW)], idx1d)

        @pl.loop(0, EW // G)
        def _(j):
            @pl.loop(0, G // 16)
            def _(k):
                idx2d[j, pl.ds(k * 16, 16)] = idx1d[pl.ds(j * G + k * 16, 16)]

        pltpu.sync_copy(zbuf, hist.at[pl.ds(s * (NP // NS), NP // NS)])
        plsc.subcore_barrier()

        @pl.loop(0, EW // G)
        def _(j):
            pltpu.sync_copy(ones, hist.at[idx2d.at[j]], add=True)

        plsc.subcore_barrier()
        pltpu.sync_copy(hist.at[pl.ds(s * (NP // NS), NP // NS)],
                        out_hbm.at[c, pl.ds(s * (NP // NS), NP // NS)])

    return deg_kernel(rows)


def _norm_body(x_ref, h0_ref, h1_ref, g_ref, bt_ref, hs_ref, dis_ref, *, bm):
    i = pl.program_id(0)
    xb = x_ref[...]
    mu = jnp.mean(xb, axis=1, keepdims=True)
    xc = xb - mu
    var = jnp.mean(xc * xc, axis=1, keepdims=True)
    h = xc * lax.rsqrt(var + 1e-5) * g_ref[...] + bt_ref[...]
    deg = h0_ref[:, 0:1] + h1_ref[:, 0:1] + 1.0
    dis = lax.rsqrt(deg)
    row = i * bm + lax.broadcasted_iota(jnp.int32, (bm, 1), 0)
    hs_ref[...] = jnp.where(row < N, h * dis, 0.0)
    dis_ref[...] = dis


def _tc_norm_scale(x_pad, h0, h1, gamma2, beta2):
    bm = 512
    return pl.pallas_call(
        functools.partial(_norm_body, bm=bm),
        grid=(NP // bm,),
        in_specs=[
            pl.BlockSpec((bm, D), lambda i: (i, 0)),
            pl.BlockSpec((bm, 16), lambda i: (i, 0)),
            pl.BlockSpec((bm, 16), lambda i: (i, 0)),
            pl.BlockSpec((1, D), lambda i: (0, 0)),
            pl.BlockSpec((1, D), lambda i: (0, 0)),
        ],
        out_specs=[
            pl.BlockSpec((bm, D), lambda i: (i, 0)),
            pl.BlockSpec((bm, 1), lambda i: (i, 0)),
        ],
        out_shape=[
            jax.ShapeDtypeStruct((NP, D), jnp.float32),
            jax.ShapeDtypeStruct((NP, 1), jnp.float32),
        ],
    )(x_pad, h0, h1, gamma2, beta2)


def _sc_aggregate(rows, cols, hs):
    @functools.partial(
        pl.kernel,
        out_type=jax.ShapeDtypeStruct((NP, D), jnp.float32),
        mesh=_MESH,
        scratch_types=[
            pltpu.VMEM((SCH,), jnp.int32),
            pltpu.VMEM((SCH,), jnp.int32),
            pltpu.VMEM((CAP,), jnp.int32),
            pltpu.VMEM((CAP,), jnp.int32),
            pltpu.VMEM((CAP // G, G), jnp.int32),
            pltpu.VMEM((G, D), jnp.float32),
            pltpu.VMEM_SHARED((H, D), jnp.float32),
            pltpu.SemaphoreType.DMA,
        ],
    )
    def agg_kernel(rows_hbm, cols_hbm, hs_hbm, out_hbm,
                   scan_r, scan_c, comp_r, comp_c, r2d, rows_data, acc, sem):
        c = lax.axis_index("c")
        s = lax.axis_index("s")
        lo = c * H

        init_cp = pltpu.async_copy(
            hs_hbm.at[pl.ds(c * H + s * RPS, RPS)],
            acc.at[pl.ds(s * RPS, RPS)], sem)

        base = s * ES

        def scan_chunk(k, ptr):
            pltpu.sync_copy(rows_hbm.at[pl.ds(base + k * SCH, SCH)], scan_r)
            pltpu.sync_copy(cols_hbm.at[pl.ds(base + k * SCH, SCH)], scan_c)

            def vec(i, ptr):
                rl = scan_r[pl.ds(i * 16, 16)] - lo
                cv = scan_c[pl.ds(i * 16, 16)]
                mask = (rl >= 0) & (rl < H)
                plsc.store_compressed(comp_r.at[pl.ds(ptr, 16)], rl, mask)
                plsc.store_compressed(comp_c.at[pl.ds(ptr, 16)], cv, mask)
                return ptr + jnp.sum(mask.astype(jnp.int32))

            return lax.fori_loop(0, SCH // 16, vec, ptr)

        ptr = lax.fori_loop(0, ES // SCH, scan_chunk, jnp.int32(0))

        pad_r = jnp.zeros((16,), jnp.int32)
        pad_c = jnp.full((16,), PAD_COL, jnp.int32)
        for k in range(G // 16):
            comp_r[pl.ds(ptr + k * 16, 16)] = pad_r
            comp_c[pl.ds(ptr + k * 16, 16)] = pad_c
        nch = (ptr + G - 1) // G

        def to2d(j, carry):
            for k in range(G // 16):
                r2d[j, pl.ds(k * 16, 16)] = comp_r[pl.ds(j * G + k * 16, 16)]
            return carry

        lax.fori_loop(0, nch, to2d, jnp.int32(0))

        init_cp.wait()
        plsc.subcore_barrier()

        def gs(j, carry):
            pltpu.sync_copy(hs_hbm.at[comp_c.at[pl.ds(j * G, G)]], rows_data)
            pltpu.sync_copy(rows_data, acc.at[r2d.at[j]], add=True)
            return carry

        lax.fori_loop(0, nch, gs, jnp.int32(0))

        plsc.subcore_barrier()
        pltpu.sync_copy(acc.at[pl.ds(s * RPS, RPS)],
                        out_hbm.at[pl.ds(c * H + s * RPS, RPS)])

    return agg_kernel(rows, cols, hs)


def _out_body(acc_ref, dis_ref, w_ref, b_ref, o_ref):
    a = acc_ref[...] * dis_ref[...]
    y = jnp.dot(a, w_ref[...], preferred_element_type=jnp.float32) + b_ref[...]
    o_ref[...] = y * 0.5 * (1.0 + lax.erf(y * (2.0 ** -0.5)))


def _tc_out(acc, dis, W, b2):
    bm = 400
    return pl.pallas_call(
        _out_body,
        grid=(N // bm,),
        in_specs=[
            pl.BlockSpec((bm, D), lambda i: (i, 0)),
            pl.BlockSpec((bm, 1), lambda i: (i, 0)),
            pl.BlockSpec((D, D), lambda i: (0, 0)),
            pl.BlockSpec((1, D), lambda i: (0, 0)),
        ],
        out_specs=pl.BlockSpec((bm, D), lambda i: (i, 0)),
        out_shape=jax.ShapeDtypeStruct((N, D), jnp.float32),
    )(acc, dis, W, b2)


def kernel(x, edge_index, W, b, ln_gamma, ln_beta):
    rows = jnp.concatenate(
        [edge_index[0], jnp.full((EP - E,), PAD_ROW, jnp.int32)])
    cols = jnp.concatenate(
        [edge_index[1], jnp.full((EP - E,), PAD_COL, jnp.int32)])
    x_pad = jnp.pad(x, ((0, NP - N), (0, 0)))

    hist = _sc_degree(rows)
    hs, dis = _tc_norm_scale(x_pad, hist[0], hist[1],
                             ln_gamma.reshape(1, D), ln_beta.reshape(1, D))
    acc = _sc_aggregate(rows, cols, hs)
    return _tc_out(acc[:N], dis[:N], W, b.reshape(1, D))


# speedup vs baseline: 10.9695x; 10.9695x over previous
import functools

import jax
import jax.numpy as jnp
from jax import lax
from jax.experimental import pallas as pl
from jax.experimental.pallas import tpu as pltpu
from jax.experimental.pallas import tpu_sc as plsc

N = 10000
E = 160000
D = 256
DC = 128

NC, NS = 2, 16
NP = 10240
EP = 163840
G = 128
NB = EP // G
BA = NB // NS
BD = NB // (NC * NS)
ZR = NP // NS
PAD_ROW = N + 100
PAD_COL = N

_CP = pltpu.CompilerParams(needs_layout_passes=False)


def _sc_mesh():
    return plsc.VectorSubcoreMesh(core_axis_name="c", subcore_axis_name="s")


def _sc_degree(rows2d):
    @functools.partial(
        pl.kernel,
        out_type=jax.ShapeDtypeStruct((NC, NP), jnp.float32),
        mesh=_sc_mesh(),
        compiler_params=_CP,
        scratch_types=[
            pltpu.VMEM((BD, G), jnp.int32),
            pltpu.VMEM((G,), jnp.float32),
            pltpu.VMEM((ZR,), jnp.float32),
            pltpu.VMEM_SHARED((NP,), jnp.float32),
        ],
    )
    def deg_kernel(rows_hbm, out_hbm, erow, ones, zeros, acc):
        c = lax.axis_index("c")
        s = lax.axis_index("s")
        ones16 = jnp.ones((16,), jnp.float32)
        zeros16 = jnp.zeros((16,), jnp.float32)

        @pl.loop(0, G // 16)
        def _(i):
            ones[pl.ds(i * 16, 16)] = ones16

        @pl.loop(0, ZR // 16)
        def _(i):
            zeros[pl.ds(i * 16, 16)] = zeros16

        pltpu.sync_copy(rows_hbm.at[pl.ds((c * NS + s) * BD, BD)], erow)
        pltpu.sync_copy(zeros, acc.at[pl.ds(s * ZR, ZR)])
        plsc.subcore_barrier()

        @pl.loop(0, BD)
        def _(j):
            pltpu.sync_copy(ones, acc.at[erow.at[j]], add=True)

        plsc.subcore_barrier()
        pltpu.sync_copy(acc.at[pl.ds(s * ZR, ZR)],
                        out_hbm.at[c].at[pl.ds(s * ZR, ZR)])

    return deg_kernel(rows2d)


def _norm_body(x_ref, hist_ref, g_ref, bt_ref, hs_ref, dis_ref, *, bm):
    i = pl.program_id(0)
    xb = x_ref[...]
    mu = jnp.mean(xb, axis=1, keepdims=True)
    xc = xb - mu
    var = jnp.mean(xc * xc, axis=1, keepdims=True)
    h = xc * lax.rsqrt(var + 1e-5) * g_ref[...] + bt_ref[...]
    deg = jnp.sum(hist_ref[...], axis=1, keepdims=True) + 1.0
    dis = lax.rsqrt(deg)
    row = i * bm + lax.broadcasted_iota(jnp.int32, (bm, 1), 0)
    hs = jnp.where(row < N, h * dis, 0.0)
    hs_ref[0] = hs[:, :DC]
    hs_ref[1] = hs[:, DC:]
    dis_ref[...] = dis


def _tc_norm_scale(x_pad, hist_t, gamma2, beta2):
    bm = 512
    return pl.pallas_call(
        functools.partial(_norm_body, bm=bm),
        grid=(NP // bm,),
        in_specs=[
            pl.BlockSpec((bm, D), lambda i: (i, 0)),
            pl.BlockSpec((bm, NC), lambda i: (i, 0)),
            pl.BlockSpec((1, D), lambda i: (0, 0)),
            pl.BlockSpec((1, D), lambda i: (0, 0)),
        ],
        out_specs=[
            pl.BlockSpec((NC, bm, DC), lambda i: (0, i, 0)),
            pl.BlockSpec((bm, 1), lambda i: (i, 0)),
        ],
        out_shape=[
            jax.ShapeDtypeStruct((NC, NP, DC), jnp.float32),
            jax.ShapeDtypeStruct((NP, 1), jnp.float32),
        ],
    )(x_pad, hist_t, gamma2, beta2)


def _sc_aggregate(rows2d, cols2d, hs2):
    @functools.partial(
        pl.kernel,
        out_type=jax.ShapeDtypeStruct((NC, NP, DC), jnp.float32),
        mesh=_sc_mesh(),
        compiler_params=_CP,
        scratch_types=[
            pltpu.VMEM((BA, G), jnp.int32),
            pltpu.VMEM((BA, G), jnp.int32),
            pltpu.VMEM((G, DC), jnp.float32),
            pltpu.VMEM_SHARED((NP, DC), jnp.float32),
        ],
    )
    def agg_kernel(rows_hbm, cols_hbm, hs_hbm, out_hbm, erow, ecol, buf, acc):
        c = lax.axis_index("c")
        s = lax.axis_index("s")

        pltpu.sync_copy(rows_hbm.at[pl.ds(s * BA, BA)], erow)
        pltpu.sync_copy(cols_hbm.at[pl.ds(s * BA, BA)], ecol)
        pltpu.sync_copy(hs_hbm.at[c].at[pl.ds(s * ZR, ZR)],
                        acc.at[pl.ds(s * ZR, ZR)])
        plsc.subcore_barrier()

        @pl.loop(0, BA)
        def _(j):
            pltpu.sync_copy(hs_hbm.at[c].at[ecol.at[j]], buf)
            pltpu.sync_copy(buf, acc.at[erow.at[j]], add=True)

        plsc.subcore_barrier()
        pltpu.sync_copy(acc.at[pl.ds(s * ZR, ZR)],
                        out_hbm.at[c].at[pl.ds(s * ZR, ZR)])

    return agg_kernel(rows2d, cols2d, hs2)


def _out_body(a_ref, dis_ref, w_ref, b_ref, o_ref):
    a = jnp.concatenate([a_ref[0], a_ref[1]], axis=1) * dis_ref[...]
    y = jnp.dot(a, w_ref[...], preferred_element_type=jnp.float32) + b_ref[...]
    o_ref[...] = y * 0.5 * (1.0 + lax.erf(y * (2.0 ** -0.5)))


def _tc_out(acc, dis, W, b2):
    bm = 400
    return pl.pallas_call(
        _out_body,
        grid=(N // bm,),
        in_specs=[
            pl.BlockSpec((NC, bm, DC), lambda i: (0, i, 0)),
            pl.BlockSpec((bm, 1), lambda i: (i, 0)),
            pl.BlockSpec((D, D), lambda i: (0, 0)),
            pl.BlockSpec((1, D), lambda i: (0, 0)),
        ],
        out_specs=pl.BlockSpec((bm, D), lambda i: (i, 0)),
        out_shape=jax.ShapeDtypeStruct((N, D), jnp.float32),
    )(acc, dis, W, b2)


def kernel(x, edge_index, W, b, ln_gamma, ln_beta):
    rows2d = jnp.concatenate(
        [edge_index[0], jnp.full((EP - E,), PAD_ROW, jnp.int32)]).reshape(NB, G)
    cols2d = jnp.concatenate(
        [edge_index[1], jnp.full((EP - E,), PAD_COL, jnp.int32)]).reshape(NB, G)
    x_pad = jnp.pad(x, ((0, NP - N), (0, 0)))

    hist = _sc_degree(rows2d)
    hs2, dis = _tc_norm_scale(x_pad, hist.T,
                              ln_gamma.reshape(1, D), ln_beta.reshape(1, D))
    acc = _sc_aggregate(rows2d, cols2d, hs2)
    return _tc_out(acc, dis, W, b.reshape(1, D))


# speedup vs baseline: 13.3562x; 1.2176x over previous
import functools

import jax
import jax.numpy as jnp
from jax import lax
from jax.experimental import pallas as pl
from jax.experimental.pallas import tpu as pltpu
from jax.experimental.pallas import tpu_sc as plsc

N = 10000
E = 160000
D = 256
DC = 128

NC, NS = 2, 16
NP = 10240
EP = 163840
G = 128
NB = EP // G
BA = NB // NS
BP = BA // 2
BD = NB // (NC * NS)
ZR = NP // NS
PAD_ROW = N + 100
PAD_COL = N

_CP = pltpu.CompilerParams(needs_layout_passes=False)


def _sc_mesh():
    return plsc.VectorSubcoreMesh(core_axis_name="c", subcore_axis_name="s")


def _sc_degree(rows2d):
    @functools.partial(
        pl.kernel,
        out_type=jax.ShapeDtypeStruct((NC, NP), jnp.float32),
        mesh=_sc_mesh(),
        compiler_params=_CP,
        scratch_types=[
            pltpu.VMEM((BD, G), jnp.int32),
            pltpu.VMEM((G,), jnp.float32),
            pltpu.VMEM((ZR,), jnp.float32),
            pltpu.VMEM_SHARED((NP,), jnp.float32),
        ],
    )
    def deg_kernel(rows_hbm, out_hbm, erow, ones, zeros, acc):
        c = lax.axis_index("c")
        s = lax.axis_index("s")
        ones16 = jnp.ones((16,), jnp.float32)
        zeros16 = jnp.zeros((16,), jnp.float32)

        @pl.loop(0, G // 16)
        def _(i):
            ones[pl.ds(i * 16, 16)] = ones16

        @pl.loop(0, ZR // 16)
        def _(i):
            zeros[pl.ds(i * 16, 16)] = zeros16

        pltpu.sync_copy(rows_hbm.at[pl.ds((c * NS + s) * BD, BD)], erow)
        pltpu.sync_copy(zeros, acc.at[pl.ds(s * ZR, ZR)])
        plsc.subcore_barrier()

        @pl.loop(0, BD)
        def _(j):
            pltpu.sync_copy(ones, acc.at[erow.at[j]], add=True)

        plsc.subcore_barrier()
        pltpu.sync_copy(acc.at[pl.ds(s * ZR, ZR)],
                        out_hbm.at[c].at[pl.ds(s * ZR, ZR)])

    return deg_kernel(rows2d)


def _norm_body(x_ref, hist_ref, g_ref, bt_ref, hs_ref, dis_ref, *, bm):
    i = pl.program_id(0)
    xb = x_ref[...]
    mu = jnp.mean(xb, axis=1, keepdims=True)
    xc = xb - mu
    var = jnp.mean(xc * xc, axis=1, keepdims=True)
    h = xc * lax.rsqrt(var + 1e-5) * g_ref[...] + bt_ref[...]
    deg = jnp.sum(hist_ref[...], axis=1, keepdims=True) + 1.0
    dis = lax.rsqrt(deg)
    row = i * bm + lax.broadcasted_iota(jnp.int32, (bm, 1), 0)
    hs = jnp.where(row < N, h * dis, 0.0)
    hs_ref[0] = hs[:, :DC]
    hs_ref[1] = hs[:, DC:]
    dis_ref[...] = dis


def _tc_norm_scale(x_pad, hist_t, gamma2, beta2):
    bm = 512
    return pl.pallas_call(
        functools.partial(_norm_body, bm=bm),
        grid=(NP // bm,),
        in_specs=[
            pl.BlockSpec((bm, D), lambda i: (i, 0)),
            pl.BlockSpec((bm, NC), lambda i: (i, 0)),
            pl.BlockSpec((1, D), lambda i: (0, 0)),
            pl.BlockSpec((1, D), lambda i: (0, 0)),
        ],
        out_specs=[
            pl.BlockSpec((NC, bm, DC), lambda i: (0, i, 0)),
            pl.BlockSpec((bm, 1), lambda i: (i, 0)),
        ],
        out_shape=[
            jax.ShapeDtypeStruct((NC, NP, DC), jnp.float32),
            jax.ShapeDtypeStruct((NP, 1), jnp.float32),
        ],
    )(x_pad, hist_t, gamma2, beta2)


def _sc_aggregate(rows2d, cols2d, hs2):
    @functools.partial(
        pl.kernel,
        out_type=jax.ShapeDtypeStruct((NC, NP, DC), jnp.float32),
        mesh=_sc_mesh(),
        compiler_params=_CP,
        scratch_types=[
            pltpu.VMEM((BP, G), jnp.int32),
            pltpu.VMEM((BP, G), jnp.int32),
            pltpu.VMEM((2, G, DC), jnp.float32),
            pltpu.VMEM_SHARED((NP, DC), jnp.float32),
            pltpu.SemaphoreType.DMA,
            pltpu.SemaphoreType.DMA,
        ],
    )
    def agg_kernel(rows_hbm, cols_hbm, hs_hbm, out_hbm,
                   erow, ecol, buf, acc, gsem, ssem):
        c = lax.axis_index("c")
        s = lax.axis_index("s")

        pltpu.sync_copy(hs_hbm.at[c].at[pl.ds(s * ZR, ZR)],
                        acc.at[pl.ds(s * ZR, ZR)])
        plsc.subcore_barrier()

        for ph in range(BA // BP):
            pltpu.sync_copy(rows_hbm.at[pl.ds(s * BA + ph * BP, BP)], erow)
            pltpu.sync_copy(cols_hbm.at[pl.ds(s * BA + ph * BP, BP)], ecol)

            pltpu.async_copy(hs_hbm.at[c].at[ecol.at[0]], buf.at[0], gsem)

            @pl.loop(0, BP)
            def _(j):
                b = lax.rem(j, 2)
                nb = 1 - b

                @pl.when(j >= 1)
                def _():
                    pltpu.make_async_copy(
                        buf.at[nb], acc.at[erow.at[j - 1]], ssem).wait()

                @pl.when(j + 1 < BP)
                def _():
                    pltpu.async_copy(
                        hs_hbm.at[c].at[ecol.at[j + 1]], buf.at[nb], gsem)

                pltpu.make_async_copy(
                    hs_hbm.at[c].at[ecol.at[j]], buf.at[b], gsem).wait()
                pltpu.async_copy(buf.at[b], acc.at[erow.at[j]], ssem, add=True)

            pltpu.make_async_copy(
                buf.at[(BP - 1) % 2], acc.at[erow.at[BP - 1]], ssem).wait()

        plsc.subcore_barrier()
        pltpu.sync_copy(acc.at[pl.ds(s * ZR, ZR)],
                        out_hbm.at[c].at[pl.ds(s * ZR, ZR)])

    return agg_kernel(rows2d, cols2d, hs2)


def _out_body(a_ref, dis_ref, w_ref, b_ref, o_ref):
    a = jnp.concatenate([a_ref[0], a_ref[1]], axis=1) * dis_ref[...]
    y = jnp.dot(a, w_ref[...], preferred_element_type=jnp.float32) + b_ref[...]
    o_ref[...] = y * 0.5 * (1.0 + lax.erf(y * (2.0 ** -0.5)))


def _tc_out(acc, dis, W, b2):
    bm = 400
    return pl.pallas_call(
        _out_body,
        grid=(N // bm,),
        in_specs=[
            pl.BlockSpec((NC, bm, DC), lambda i: (0, i, 0)),
            pl.BlockSpec((bm, 1), lambda i: (i, 0)),
            pl.BlockSpec((D, D), lambda i: (0, 0)),
            pl.BlockSpec((1, D), lambda i: (0, 0)),
        ],
        out_specs=pl.BlockSpec((bm, D), lambda i: (i, 0)),
        out_shape=jax.ShapeDtypeStruct((N, D), jnp.float32),
    )(acc, dis, W, b2)


def kernel(x, edge_index, W, b, ln_gamma, ln_beta):
    rows2d = jnp.concatenate(
        [edge_index[0], jnp.full((EP - E,), PAD_ROW, jnp.int32)]).reshape(NB, G)
    cols2d = jnp.concatenate(
        [edge_index[1], jnp.full((EP - E,), PAD_COL, jnp.int32)]).reshape(NB, G)
    x_pad = jnp.pad(x, ((0, NP - N), (0, 0)))

    hist = _sc_degree(rows2d)
    hs2, dis = _tc_norm_scale(x_pad, hist.T,
                              ln_gamma.reshape(1, D), ln_beta.reshape(1, D))
    acc = _sc_aggregate(rows2d, cols2d, hs2)
    return _tc_out(acc, dis, W, b.reshape(1, D))
